# Initial kernel scaffold; baseline (speedup 1.0000x reference)
import functools

import jax
import jax.numpy as jnp
from jax import lax
from jax.experimental import pallas as pl
from jax.experimental.pallas import tpu as pltpu
from jax.experimental.pallas import tpu_sc as plsc

N = 10000
D = 128
E = 320000

NC, NS, L = 2, 16, 16
NW = NC * NS

CHUNK = 128
EPT = ((E + NW * CHUNK - 1) // (NW * CHUNK)) * CHUNK
EPAD = EPT * NW
NPAD = ((N + NS * L - 1) // (NS * L)) * NS * L
RPT = NPAD // NS

_mesh = plsc.VectorSubcoreMesh(core_axis_name="c", subcore_axis_name="s")


def _deg_body(col_hbm, zeros_hbm, ones_hbm, degp_hbm, idx_v, ones_v, acc):
    c = lax.axis_index("c")
    s = lax.axis_index("s")
    wid = c * NS + s
    pltpu.sync_copy(zeros_hbm, acc.at[pl.ds(s * RPT, RPT)])
    pltpu.sync_copy(ones_hbm, ones_v)
    plsc.subcore_barrier()
    base = wid * EPT

    def body(k, carry):
        pltpu.sync_copy(col_hbm.at[pl.ds(base + k * CHUNK, CHUNK)], idx_v)
        pltpu.sync_copy(ones_v, acc.at[idx_v], add=True)
        return carry

    lax.fori_loop(0, EPT // CHUNK, body, 0)
    plsc.subcore_barrier()
    pltpu.sync_copy(acc.at[pl.ds(s * RPT, RPT)], degp_hbm.at[c, pl.ds(s * RPT, RPT)])


_deg_call = pl.kernel(
    _deg_body,
    out_type=jax.ShapeDtypeStruct((NC, NPAD, L), jnp.float32),
    mesh=_mesh,
    scratch_types=[
        pltpu.VMEM((CHUNK,), jnp.int32),
        pltpu.VMEM((CHUNK, L), jnp.float32),
        pltpu.VMEM_SHARED((NPAD, L), jnp.float32),
    ],
)


def _agg_body(hs_hbm, row_hbm, col_hbm, zeros_hbm, part_hbm, ridx, cidx, gbuf, acc, sem):
    c = lax.axis_index("c")
    s = lax.axis_index("s")
    wid = c * NS + s
    pltpu.sync_copy(zeros_hbm, acc.at[pl.ds(s * RPT, RPT)])
    plsc.subcore_barrier()
    base = wid * EPT

    def body(k, carry):
        off = base + k * CHUNK
        pltpu.sync_copy(row_hbm.at[pl.ds(off, CHUNK)], ridx)
        pltpu.sync_copy(col_hbm.at[pl.ds(off, CHUNK)], cidx)
        pltpu.async_copy(hs_hbm.at[ridx], gbuf, sem).wait()
        pltpu.sync_copy(gbuf, acc.at[cidx], add=True)
        return carry

    lax.fori_loop(0, EPT // CHUNK, body, 0)
    plsc.subcore_barrier()
    pltpu.sync_copy(acc.at[pl.ds(s * RPT, RPT)], part_hbm.at[c, pl.ds(s * RPT, RPT)])


_agg_call = pl.kernel(
    _agg_body,
    out_type=jax.ShapeDtypeStruct((NC, NPAD, D), jnp.float32),
    mesh=_mesh,
    scratch_types=[
        pltpu.VMEM((CHUNK,), jnp.int32),
        pltpu.VMEM((CHUNK,), jnp.int32),
        pltpu.VMEM((CHUNK, D), jnp.float32),
        pltpu.VMEM_SHARED((NPAD, D), jnp.float32),
        pltpu.SemaphoreType.DMA,
    ],
)


BM = 1024


def _mm_body(x_ref, w_ref, degp_ref, hs_ref):
    d = degp_ref[0] + degp_ref[1]
    dis = lax.rsqrt(d[:, 0:1] + 1.0)
    h = jnp.dot(x_ref[...], w_ref[...], preferred_element_type=jnp.float32)
    hs_ref[...] = h * dis


def _ep_body(part_ref, hs_ref, degp_ref, b_ref, out_ref):
    d = degp_ref[0] + degp_ref[1]
    dis = lax.rsqrt(d[:, 0:1] + 1.0)
    agg = part_ref[0] + part_ref[1] + hs_ref[...]
    out_ref[...] = jnp.maximum(agg * dis + b_ref[...], 0.0)


def kernel(x, edge_index, W, b):
    row = edge_index[0].astype(jnp.int32)
    col = edge_index[1].astype(jnp.int32)
    npad_extra = NPAD - N
    pad = EPAD - E
    row_p = jnp.concatenate([row, jnp.zeros((pad,), jnp.int32)])
    col_p = jnp.concatenate(
        [col, N + (jnp.arange(pad, dtype=jnp.int32) % npad_extra)]
    )
    x_p = jnp.concatenate([x, jnp.zeros((NPAD - N, D), x.dtype)])

    zeros16 = jnp.zeros((RPT, L), jnp.float32)
    ones16 = jnp.ones((CHUNK, L), jnp.float32)
    zerosD = jnp.zeros((RPT, D), jnp.float32)

    degp = _deg_call(col_p, zeros16, ones16)

    hs = pl.pallas_call(
        _mm_body,
        grid=(NPAD // BM,),
        in_specs=[
            pl.BlockSpec((BM, D), lambda i: (i, 0)),
            pl.BlockSpec((D, D), lambda i: (0, 0)),
            pl.BlockSpec((NC, BM, L), lambda i: (0, i, 0)),
        ],
        out_specs=pl.BlockSpec((BM, D), lambda i: (i, 0)),
        out_shape=jax.ShapeDtypeStruct((NPAD, D), jnp.float32),
    )(x_p, W, degp)

    part = _agg_call(hs, row_p, col_p, zerosD)

    out = pl.pallas_call(
        _ep_body,
        grid=(NPAD // BM,),
        in_specs=[
            pl.BlockSpec((NC, BM, D), lambda i: (0, i, 0)),
            pl.BlockSpec((BM, D), lambda i: (i, 0)),
            pl.BlockSpec((NC, BM, L), lambda i: (0, i, 0)),
            pl.BlockSpec((1, D), lambda i: (0, 0)),
        ],
        out_specs=pl.BlockSpec((BM, D), lambda i: (i, 0)),
        out_shape=jax.ShapeDtypeStruct((NPAD, D), jnp.float32),
    )(part, hs, degp, b.reshape(1, D))

    return out[:N]


# speedup vs baseline: 14.7645x; 14.7645x over previous
import functools

import jax
import jax.numpy as jnp
from jax import lax
from jax.experimental import pallas as pl
from jax.experimental.pallas import tpu as pltpu
from jax.experimental.pallas import tpu_sc as plsc

N = 10000
D = 128
E = 320000

NC, NS, L = 2, 16, 16
NW = NC * NS

CHUNK = 128
EPT = ((E + NW * CHUNK - 1) // (NW * CHUNK)) * CHUNK
EPAD = EPT * NW
NPAD = ((N + NS * L - 1) // (NS * L)) * NS * L
RPT = NPAD // NS

def _deg_body(col_hbm, zeros_hbm, degp_hbm, idx_v, acc):
    c = lax.axis_index("c")
    s = lax.axis_index("s")
    wid = c * NS + s
    pltpu.sync_copy(zeros_hbm, acc)
    base = wid * EPT
    ones = jnp.ones((L,), jnp.float32)

    def chunk(k, carry):
        pltpu.sync_copy(col_hbm.at[pl.ds(base + k * CHUNK, CHUNK)], idx_v)

        def inner(j, cc):
            idx16 = idx_v[pl.ds(j * L, L)]
            plsc.addupdate_scatter(acc, [idx16], ones)
            return cc

        lax.fori_loop(0, CHUNK // L, inner, 0)
        return carry

    lax.fori_loop(0, EPT // CHUNK, chunk, 0)
    pltpu.sync_copy(acc, degp_hbm.at[wid])


def _agg_body(hs_hbm, row_hbm, col_hbm, zeros_hbm, part_hbm, ridx, cidx, gbuf, acc, sem):
    c = lax.axis_index("c")
    s = lax.axis_index("s")
    wid = c * NS + s
    pltpu.sync_copy(zeros_hbm, acc.at[pl.ds(s * RPT, RPT)])
    plsc.subcore_barrier()
    base = wid * EPT

    def body(k, carry):
        off = base + k * CHUNK
        pltpu.sync_copy(row_hbm.at[pl.ds(off, CHUNK)], ridx)
        pltpu.sync_copy(col_hbm.at[pl.ds(off, CHUNK)], cidx)
        pltpu.async_copy(hs_hbm.at[ridx], gbuf, sem).wait()
        pltpu.sync_copy(gbuf, acc.at[cidx], add=True)
        return carry

    lax.fori_loop(0, EPT // CHUNK, body, 0)
    plsc.subcore_barrier()
    pltpu.sync_copy(acc.at[pl.ds(s * RPT, RPT)], part_hbm.at[c, pl.ds(s * RPT, RPT)])


@functools.lru_cache(maxsize=None)
def _sc_calls():
    mesh = plsc.VectorSubcoreMesh(
        core_axis_name="c", subcore_axis_name="s", num_cores=NC, num_subcores=NS
    )
    deg_call = pl.kernel(
        _deg_body,
        out_type=jax.ShapeDtypeStruct((NW, NPAD), jnp.float32),
        mesh=mesh,
        scratch_types=[
            pltpu.VMEM((CHUNK,), jnp.int32),
            pltpu.VMEM((NPAD,), jnp.float32),
        ],
        compiler_params=pltpu.CompilerParams(needs_layout_passes=False),
    )
    agg_call = pl.kernel(
        _agg_body,
        out_type=jax.ShapeDtypeStruct((NC, NPAD, D), jnp.float32),
        mesh=mesh,
        scratch_types=[
            pltpu.VMEM((CHUNK,), jnp.int32),
            pltpu.VMEM((CHUNK,), jnp.int32),
            pltpu.VMEM((CHUNK, D), jnp.float32),
            pltpu.VMEM_SHARED((NPAD, D), jnp.float32),
            pltpu.SemaphoreType.DMA,
        ],
    )
    return deg_call, agg_call


BM = 1024


def _dis_col(degp_block):
    ones = jnp.ones((NW, 1), jnp.float32)
    d = lax.dot_general(
        degp_block, ones, (((0,), (0,)), ((), ())),
        preferred_element_type=jnp.float32,
    )
    return lax.rsqrt(d + 1.0)


def _mm_body(x_ref, w_ref, degp_ref, hs_ref):
    dis = _dis_col(degp_ref[...])
    h = jnp.dot(x_ref[...], w_ref[...], preferred_element_type=jnp.float32)
    hs_ref[...] = h * dis


def _ep_body(part_ref, hs_ref, degp_ref, b_ref, out_ref):
    dis = _dis_col(degp_ref[...])
    agg = part_ref[0] + part_ref[1] + hs_ref[...]
    out_ref[...] = jnp.maximum(agg * dis + b_ref[...], 0.0)


def kernel(x, edge_index, W, b):
    row = edge_index[0].astype(jnp.int32)
    col = edge_index[1].astype(jnp.int32)
    npad_extra = NPAD - N
    pad = EPAD - E
    row_p = jnp.concatenate([row, jnp.zeros((pad,), jnp.int32)])
    col_p = jnp.concatenate(
        [col, N + (jnp.arange(pad, dtype=jnp.int32) % npad_extra)]
    )
    x_p = jnp.concatenate([x, jnp.zeros((NPAD - N, D), x.dtype)])

    zerosN = jnp.zeros((NPAD,), jnp.float32)
    zerosD = jnp.zeros((RPT, D), jnp.float32)

    deg_call, agg_call = _sc_calls()
    degp = deg_call(col_p, zerosN)

    hs = pl.pallas_call(
        _mm_body,
        grid=(NPAD // BM,),
        in_specs=[
            pl.BlockSpec((BM, D), lambda i: (i, 0)),
            pl.BlockSpec((D, D), lambda i: (0, 0)),
            pl.BlockSpec((NW, BM), lambda i: (0, i)),
        ],
        out_specs=pl.BlockSpec((BM, D), lambda i: (i, 0)),
        out_shape=jax.ShapeDtypeStruct((NPAD, D), jnp.float32),
    )(x_p, W, degp)

    part = agg_call(hs, row_p, col_p, zerosD)

    out = pl.pallas_call(
        _ep_body,
        grid=(NPAD // BM,),
        in_specs=[
            pl.BlockSpec((NC, BM, D), lambda i: (0, i, 0)),
            pl.BlockSpec((BM, D), lambda i: (i, 0)),
            pl.BlockSpec((NW, BM), lambda i: (0, i)),
            pl.BlockSpec((1, D), lambda i: (0, 0)),
        ],
        out_specs=pl.BlockSpec((BM, D), lambda i: (i, 0)),
        out_shape=jax.ShapeDtypeStruct((NPAD, D), jnp.float32),
    )(part, hs, degp, b.reshape(1, D))

    return out[:N]


# speedup vs baseline: 42.6275x; 2.8872x over previous
import functools

import jax
import jax.numpy as jnp
from jax import lax
from jax.experimental import pallas as pl
from jax.experimental.pallas import tpu as pltpu
from jax.experimental.pallas import tpu_sc as plsc

N = 10000
D = 128
E = 320000

NC, NS, L = 2, 16, 16
NW = NC * NS

CHUNK = 128
NBUF = 2
NRING = 2 * NBUF
EPT = ((E + NW * CHUNK * NRING - 1) // (NW * CHUNK * NRING)) * CHUNK * NRING
EPAD = EPT * NW
NCH = EPT // CHUNK
NPAD = ((N + NS * L - 1) // (NS * L)) * NS * L
RSTRIDE = 624
RWIN = 640
assert RSTRIDE * (NS - 1) + RWIN == N


def _deg_body(col_hbm, zeros_hbm, degp_hbm, idx_v, acc):
    c = lax.axis_index("c")
    s = lax.axis_index("s")
    wid = c * NS + s
    pltpu.sync_copy(zeros_hbm, acc)
    pltpu.sync_copy(col_hbm.at[wid], idx_v)
    ones = jnp.ones((L,), jnp.float32)

    def chunk(k, carry):
        for j in range(CHUNK // L):
            idx16 = idx_v[k, pl.ds(j * L, L)]
            plsc.addupdate_scatter(acc, [idx16], ones)
        return carry

    lax.fori_loop(0, NCH, chunk, 0)
    pltpu.sync_copy(acc, degp_hbm.at[wid])


def _agg_body(
    hs_hbm, edges_hbm, zeros_hbm, part_hbm,
    acc, eb0, eb1, eb2, eb3, gb0, gb1, se0, se1, se2, se3, sg0, sg1
):
    c = lax.axis_index("c")
    s = lax.axis_index("s")
    wid = c * NS + s
    ebufs = (eb0, eb1, eb2, eb3)
    esems = (se0, se1, se2, se3)
    gbufs = (gb0, gb1)
    gsems = (sg0, sg1)

    pltpu.sync_copy(zeros_hbm, acc.at[pl.ds(s * RSTRIDE, RWIN)])
    for b in range(NBUF):
        pltpu.sync_copy(edges_hbm.at[wid, b], ebufs[b])
    plsc.subcore_barrier()
    for b in range(NBUF):
        pltpu.async_copy(hs_hbm.at[ebufs[b].at[0]], gbufs[b], gsems[b])
    for k in range(NBUF, NRING):
        pltpu.async_copy(edges_hbm.at[wid, k], ebufs[k], esems[k])

    def outer(g, carry):
        for b in range(NRING):
            k = g * NRING + b
            gb = b % NBUF
            pltpu.make_async_copy(hs_hbm.at[pl.ds(0, CHUNK)], gbufs[gb], gsems[gb]).wait()
            pltpu.sync_copy(gbufs[gb], acc.at[ebufs[b].at[1]], add=True)

            @pl.when(k + NBUF < NCH)
            def _():
                nb = (b + NBUF) % NRING
                pltpu.make_async_copy(edges_hbm.at[wid, 0], ebufs[nb], esems[nb]).wait()
                pltpu.async_copy(hs_hbm.at[ebufs[nb].at[0]], gbufs[gb], gsems[gb])

            @pl.when(k + NRING < NCH)
            def _():
                pltpu.async_copy(edges_hbm.at[wid, k + NRING], ebufs[b], esems[b])

        return carry

    lax.fori_loop(0, NCH // NRING, outer, 0)
    plsc.subcore_barrier()
    pltpu.sync_copy(
        acc.at[pl.ds(s * RSTRIDE, RWIN)], part_hbm.at[c, pl.ds(s * RSTRIDE, RWIN)]
    )


@functools.lru_cache(maxsize=None)
def _sc_calls():
    mesh = plsc.VectorSubcoreMesh(
        core_axis_name="c", subcore_axis_name="s", num_cores=NC, num_subcores=NS
    )
    deg_call = pl.kernel(
        _deg_body,
        out_type=jax.ShapeDtypeStruct((NW, NPAD), jnp.float32),
        mesh=mesh,
        scratch_types=[
            pltpu.VMEM((NCH, CHUNK), jnp.int32),
            pltpu.VMEM((NPAD,), jnp.float32),
        ],
        compiler_params=pltpu.CompilerParams(needs_layout_passes=False),
    )
    agg_call = pl.kernel(
        _agg_body,
        out_type=jax.ShapeDtypeStruct((NC, NPAD, D), jnp.float32),
        mesh=mesh,
        scratch_types=[pltpu.VMEM_SHARED((N, D), jnp.float32)]
        + [pltpu.VMEM((2, CHUNK), jnp.int32)] * NRING
        + [pltpu.VMEM((CHUNK, D), jnp.float32)] * NBUF
        + [pltpu.SemaphoreType.DMA] * (NRING + NBUF),
    )
    return deg_call, agg_call


BM = 1024


def _dis_col(degp_block):
    ones = jnp.ones((NW, 1), jnp.float32)
    d = lax.dot_general(
        degp_block, ones, (((0,), (0,)), ((), ())),
        preferred_element_type=jnp.float32,
    )
    return lax.rsqrt(d + 1.0)


def _mm_body(x_ref, w_ref, degp_ref, hs_ref):
    dis = _dis_col(degp_ref[...])
    h = jnp.dot(x_ref[...], w_ref[...], preferred_element_type=jnp.float32)
    hs_ref[...] = h * dis


def _ep_body(part_ref, hs_ref, degp_ref, b_ref, out_ref):
    dis = _dis_col(degp_ref[...])
    agg = part_ref[0] + part_ref[1] + hs_ref[...]
    out_ref[...] = jnp.maximum(agg * dis + b_ref[...], 0.0)


def kernel(x, edge_index, W, b):
    row = edge_index[0].astype(jnp.int32)
    col = edge_index[1].astype(jnp.int32)
    pad = EPAD - E
    arange_pad = jnp.arange(pad, dtype=jnp.int32)
    row_a = jnp.concatenate([row, N + arange_pad % (NPAD - N)])
    col_a = jnp.concatenate([col, arange_pad % N])
    col_d = jnp.concatenate([col, N + arange_pad % (NPAD - N)])
    col_d3 = col_d.reshape(NW, NCH, CHUNK)
    edges3 = jnp.stack(
        [row_a.reshape(NW, NCH, CHUNK), col_a.reshape(NW, NCH, CHUNK)], axis=2
    )
    x_p = jnp.concatenate([x, jnp.zeros((NPAD - N, D), x.dtype)])

    zerosN = jnp.zeros((NPAD,), jnp.float32)
    zerosD = jnp.zeros((RWIN, D), jnp.float32)

    deg_call, agg_call = _sc_calls()
    degp = deg_call(col_d3, zerosN)

    hs = pl.pallas_call(
        _mm_body,
        grid=(NPAD // BM,),
        in_specs=[
            pl.BlockSpec((BM, D), lambda i: (i, 0)),
            pl.BlockSpec((D, D), lambda i: (0, 0)),
            pl.BlockSpec((NW, BM), lambda i: (0, i)),
        ],
        out_specs=pl.BlockSpec((BM, D), lambda i: (i, 0)),
        out_shape=jax.ShapeDtypeStruct((NPAD, D), jnp.float32),
    )(x_p, W, degp)

    part = agg_call(hs, edges3, zerosD)

    out = pl.pallas_call(
        _ep_body,
        grid=(NPAD // BM,),
        in_specs=[
            pl.BlockSpec((NC, BM, D), lambda i: (0, i, 0)),
            pl.BlockSpec((BM, D), lambda i: (i, 0)),
            pl.BlockSpec((NW, BM), lambda i: (0, i)),
            pl.BlockSpec((1, D), lambda i: (0, 0)),
        ],
        out_specs=pl.BlockSpec((BM, D), lambda i: (i, 0)),
        out_shape=jax.ShapeDtypeStruct((NPAD, D), jnp.float32),
    )(part, hs, degp, b.reshape(1, D))

    return out[:N]
